# Initial kernel scaffold; baseline (speedup 1.0000x reference)
import jax
import jax.numpy as jnp
from jax.experimental import pallas as pl


def kernel(x, edge_index, Wl0, bl0, Wr0, Wl1, bl1, Wr1, Wl2, bl2, Wr2, Wp, bp, gamma, beta):
    raise NotImplementedError("write your pallas kernel here")


# speedup vs baseline: 4.5942x; 4.5942x over previous
import jax
import jax.numpy as jnp
from jax import lax
from jax.experimental import pallas as pl
from jax.experimental.pallas import tpu as pltpu
from jax.experimental.pallas import tpu_sc as plsc

N = 10000
E = 320000
D = 128

NC = 2
NS = 16
NW = NC * NS
EPW = E // NW
CHUNK = 80
NCHUNK = EPW // CHUNK
RPS = 624
TAIL = N - NS * RPS

_F32 = jnp.float32
_MESH = plsc.VectorSubcoreMesh(core_axis_name="c", subcore_axis_name="s")


def _agg_body(x_hbm, src_hbm, dst_hbm, acc_out, src_v, dst_v, rows_v, acc_s):
  cid = lax.axis_index("c")
  sid = lax.axis_index("s")
  wid = sid * NC + cid

  @pl.loop(0, CHUNK)
  def _(i):
    @pl.loop(0, D // 16)
    def _(j):
      rows_v[i, pl.ds(j * 16, 16)] = jnp.zeros((16,), _F32)

  n_full = RPS // CHUNK
  rem = RPS - n_full * CHUNK
  row0 = sid * RPS

  @pl.loop(0, n_full)
  def _(k):
    pltpu.sync_copy(rows_v, acc_s.at[pl.ds(row0 + k * CHUNK, CHUNK)])
  pltpu.sync_copy(rows_v.at[pl.ds(0, rem)],
                  acc_s.at[pl.ds(row0 + n_full * CHUNK, rem)])

  @pl.when(sid == NS - 1)
  def _():
    pltpu.sync_copy(rows_v.at[pl.ds(0, TAIL)], acc_s.at[pl.ds(NS * RPS, TAIL)])

  plsc.subcore_barrier()

  base = wid * EPW

  @pl.loop(0, NCHUNK)
  def _(k):
    off = base + k * CHUNK
    pltpu.sync_copy(src_hbm.at[pl.ds(off, CHUNK)], src_v)
    pltpu.sync_copy(dst_hbm.at[pl.ds(off, CHUNK)], dst_v)
    pltpu.sync_copy(x_hbm.at[src_v], rows_v)
    pltpu.sync_copy(rows_v, acc_s.at[dst_v], add=True)

  plsc.subcore_barrier()

  @pl.loop(0, n_full)
  def _(k):
    r = row0 + k * CHUNK
    pltpu.sync_copy(acc_s.at[pl.ds(r, CHUNK)], rows_v)
    pltpu.sync_copy(rows_v, acc_out.at[cid, pl.ds(r, CHUNK)])
  r = row0 + n_full * CHUNK
  pltpu.sync_copy(acc_s.at[pl.ds(r, rem)], rows_v.at[pl.ds(0, rem)])
  pltpu.sync_copy(rows_v.at[pl.ds(0, rem)], acc_out.at[cid, pl.ds(r, rem)])

  @pl.when(sid == NS - 1)
  def _():
    pltpu.sync_copy(acc_s.at[pl.ds(NS * RPS, TAIL)], rows_v.at[pl.ds(0, TAIL)])
    pltpu.sync_copy(rows_v.at[pl.ds(0, TAIL)],
                    acc_out.at[cid, pl.ds(NS * RPS, TAIL)])


_agg = pl.kernel(
    _agg_body,
    out_type=jax.ShapeDtypeStruct((NC, N, D), _F32),
    mesh=_MESH,
    scratch_types=[
        pltpu.VMEM((CHUNK,), jnp.int32),
        pltpu.VMEM((CHUNK,), jnp.int32),
        pltpu.VMEM((CHUNK, D), _F32),
        pltpu.VMEM_SHARED((N, D), _F32),
    ],
)


def _cnt_body(dst_hbm, cnt_out, dst_v, ones_v, cnt_s):
  cid = lax.axis_index("c")
  sid = lax.axis_index("s")
  wid = sid * NC + cid

  @pl.loop(0, CHUNK)
  def _(i):
    @pl.loop(0, D // 16)
    def _(j):
      ones_v[i, pl.ds(j * 16, 16)] = jnp.zeros((16,), _F32)

  n_full = RPS // CHUNK
  rem = RPS - n_full * CHUNK
  row0 = sid * RPS

  @pl.loop(0, n_full)
  def _(k):
    pltpu.sync_copy(ones_v, cnt_s.at[pl.ds(row0 + k * CHUNK, CHUNK)])
  pltpu.sync_copy(ones_v.at[pl.ds(0, rem)],
                  cnt_s.at[pl.ds(row0 + n_full * CHUNK, rem)])

  @pl.when(sid == NS - 1)
  def _():
    pltpu.sync_copy(ones_v.at[pl.ds(0, TAIL)], cnt_s.at[pl.ds(NS * RPS, TAIL)])

  @pl.loop(0, CHUNK)
  def _(i):
    @pl.loop(0, D // 16)
    def _(j):
      ones_v[i, pl.ds(j * 16, 16)] = jnp.full((16,), 1.0, _F32)

  plsc.subcore_barrier()

  base = wid * EPW

  @pl.loop(0, NCHUNK)
  def _(k):
    pltpu.sync_copy(dst_hbm.at[pl.ds(base + k * CHUNK, CHUNK)], dst_v)
    pltpu.sync_copy(ones_v, cnt_s.at[dst_v], add=True)

  plsc.subcore_barrier()

  @pl.loop(0, n_full)
  def _(k):
    r = row0 + k * CHUNK
    pltpu.sync_copy(cnt_s.at[pl.ds(r, CHUNK)], ones_v)
    pltpu.sync_copy(ones_v, cnt_out.at[cid, pl.ds(r, CHUNK)])
  r = row0 + n_full * CHUNK
  pltpu.sync_copy(cnt_s.at[pl.ds(r, rem)], ones_v.at[pl.ds(0, rem)])
  pltpu.sync_copy(ones_v.at[pl.ds(0, rem)], cnt_out.at[cid, pl.ds(r, rem)])

  @pl.when(sid == NS - 1)
  def _():
    pltpu.sync_copy(cnt_s.at[pl.ds(NS * RPS, TAIL)], ones_v.at[pl.ds(0, TAIL)])
    pltpu.sync_copy(ones_v.at[pl.ds(0, TAIL)],
                    cnt_out.at[cid, pl.ds(NS * RPS, TAIL)])


_counts = pl.kernel(
    _cnt_body,
    out_type=jax.ShapeDtypeStruct((NC, N, D), _F32),
    mesh=_MESH,
    scratch_types=[
        pltpu.VMEM((CHUNK,), jnp.int32),
        pltpu.VMEM((CHUNK, D), _F32),
        pltpu.VMEM_SHARED((N, D), _F32),
    ],
)


def _dot_t(a, w):
  return lax.dot_general(a, w, (((1,), (1,)), ((), ())),
                         precision=lax.Precision.HIGHEST,
                         preferred_element_type=_F32)


_BR = 1000


def _layer_tc(acc, rec, h, Wl, bl, Wr, relu):
  def body(acc_ref, rec_ref, h_ref, wl_ref, bl_ref, wr_ref, out_ref):
    mean = (acc_ref[0] + acc_ref[1]) * rec_ref[...]
    y = _dot_t(mean, wl_ref[...]) + _dot_t(h_ref[...], wr_ref[...]) + bl_ref[...]
    out_ref[...] = jnp.maximum(y, 0.0) if relu else y

  return pl.pallas_call(
      body,
      grid=(N // _BR,),
      in_specs=[
          pl.BlockSpec((NC, _BR, D), lambda i: (0, i, 0)),
          pl.BlockSpec((_BR, 1), lambda i: (i, 0)),
          pl.BlockSpec((_BR, D), lambda i: (i, 0)),
          pl.BlockSpec((D, D), lambda i: (0, 0)),
          pl.BlockSpec((1, D), lambda i: (0, 0)),
          pl.BlockSpec((D, D), lambda i: (0, 0)),
      ],
      out_specs=pl.BlockSpec((_BR, D), lambda i: (i, 0)),
      out_shape=jax.ShapeDtypeStruct((N, D), _F32),
  )(acc, rec, h, Wl, bl, Wr)


def _recip_tc(cnt):
  def body(cnt_ref, out_ref):
    c = cnt_ref[0, :, 0:1] + cnt_ref[1, :, 0:1]
    out_ref[...] = 1.0 / jnp.maximum(c, 1.0)

  return pl.pallas_call(
      body,
      grid=(N // _BR,),
      in_specs=[pl.BlockSpec((NC, _BR, D), lambda i: (0, i, 0))],
      out_specs=pl.BlockSpec((_BR, 1), lambda i: (i, 0)),
      out_shape=jax.ShapeDtypeStruct((N, 1), _F32),
  )(cnt)


def _last_layer_tc(acc, rec, h, Wl, bl, Wr, Wp, bp):
  g = N // _BR

  def body(acc_ref, rec_ref, h_ref, wl_ref, bl_ref, wr_ref, wp_ref, bp_ref,
           h3_ref, proj_ref, stats_ref, ssum, ssq):
    i = pl.program_id(0)
    mean = (acc_ref[0] + acc_ref[1]) * rec_ref[...]
    y = _dot_t(mean, wl_ref[...]) + _dot_t(h_ref[...], wr_ref[...]) + bl_ref[...]
    h3_ref[...] = y
    p = _dot_t(y, wp_ref[...]) + bp_ref[...]
    proj_ref[...] = p

    @pl.when(i == 0)
    def _():
      ssum[...] = jnp.zeros_like(ssum)
      ssq[...] = jnp.zeros_like(ssq)

    ssum[...] += jnp.sum(p, axis=0, keepdims=True)
    ssq[...] += jnp.sum(p * p, axis=0, keepdims=True)

    @pl.when(i == g - 1)
    def _():
      stats_ref[0:1, :] = ssum[...]
      stats_ref[1:2, :] = ssq[...]

  return pl.pallas_call(
      body,
      grid=(g,),
      in_specs=[
          pl.BlockSpec((NC, _BR, D), lambda i: (0, i, 0)),
          pl.BlockSpec((_BR, 1), lambda i: (i, 0)),
          pl.BlockSpec((_BR, D), lambda i: (i, 0)),
          pl.BlockSpec((D, D), lambda i: (0, 0)),
          pl.BlockSpec((1, D), lambda i: (0, 0)),
          pl.BlockSpec((D, D), lambda i: (0, 0)),
          pl.BlockSpec((D, D), lambda i: (0, 0)),
          pl.BlockSpec((1, D), lambda i: (0, 0)),
      ],
      out_specs=[
          pl.BlockSpec((_BR, D), lambda i: (i, 0)),
          pl.BlockSpec((_BR, D), lambda i: (i, 0)),
          pl.BlockSpec((2, D), lambda i: (0, 0)),
      ],
      out_shape=[
          jax.ShapeDtypeStruct((N, D), _F32),
          jax.ShapeDtypeStruct((N, D), _F32),
          jax.ShapeDtypeStruct((2, D), _F32),
      ],
      scratch_shapes=[pltpu.VMEM((1, D), _F32), pltpu.VMEM((1, D), _F32)],
  )(acc, rec, h, Wl, bl, Wr, Wp, bp)


def _bnorm_tc(proj, stats, gamma, beta):
  def body(proj_ref, stats_ref, g_ref, b_ref, out_ref):
    inv_n = 1.0 / N
    mu = stats_ref[0:1, :] * inv_n
    var = stats_ref[1:2, :] * inv_n - mu * mu
    scale = g_ref[...] * lax.rsqrt(var + 1e-5)
    out_ref[...] = proj_ref[...] * scale + (b_ref[...] - mu * scale)

  return pl.pallas_call(
      body,
      grid=(N // _BR,),
      in_specs=[
          pl.BlockSpec((_BR, D), lambda i: (i, 0)),
          pl.BlockSpec((2, D), lambda i: (0, 0)),
          pl.BlockSpec((1, D), lambda i: (0, 0)),
          pl.BlockSpec((1, D), lambda i: (0, 0)),
      ],
      out_specs=pl.BlockSpec((_BR, D), lambda i: (i, 0)),
      out_shape=jax.ShapeDtypeStruct((N, D), _F32),
  )(proj, stats, gamma, beta)


def kernel(x, edge_index, Wl0, bl0, Wr0, Wl1, bl1, Wr1, Wl2, bl2, Wr2,
           Wp, bp, gamma, beta):
  src = edge_index[0]
  dst = edge_index[1]
  bl0r = bl0.reshape(1, D)
  bl1r = bl1.reshape(1, D)
  bl2r = bl2.reshape(1, D)
  bpr = bp.reshape(1, D)
  gr = gamma.reshape(1, D)
  br = beta.reshape(1, D)

  cnt = _counts(dst)
  rec = _recip_tc(cnt)
  acc0 = _agg(x, src, dst)
  h1 = _layer_tc(acc0, rec, x, Wl0, bl0r, Wr0, True)
  acc1 = _agg(h1, src, dst)
  h2 = _layer_tc(acc1, rec, h1, Wl1, bl1r, Wr1, True)
  acc2 = _agg(h2, src, dst)
  h3, proj, stats = _last_layer_tc(acc2, rec, h2, Wl2, bl2r, Wr2, Wp, bpr)
  h_out = _bnorm_tc(proj, stats, gr, br)
  return (h3, h_out)


# speedup vs baseline: 5.2835x; 1.1500x over previous
import jax
import jax.numpy as jnp
from jax import lax
from jax.experimental import pallas as pl
from jax.experimental.pallas import tpu as pltpu
from jax.experimental.pallas import tpu_sc as plsc

N = 10000
E = 320000
D = 128

NC = 2
NS = 16
NW = NC * NS
EPW = E // NW
CHUNK = 80
NCHUNK = EPW // CHUNK
RPS = 624
TAIL = N - NS * RPS

_F32 = jnp.float32
_MESH = plsc.VectorSubcoreMesh(core_axis_name="c", subcore_axis_name="s")


NBUF = 4
N2 = NCHUNK // NBUF


def _agg_body(x_hbm, src_hbm, dst_hbm, acc_out, src_v, dst_v, rows_v, acc_s,
              *sems):
  cid = lax.axis_index("c")
  sid = lax.axis_index("s")
  wid = sid * NC + cid
  z = rows_v.at[0]

  @pl.loop(0, CHUNK)
  def _(i):
    @pl.loop(0, D // 16)
    def _(j):
      rows_v[0, i, pl.ds(j * 16, 16)] = jnp.zeros((16,), _F32)

  n_full = RPS // CHUNK
  rem = RPS - n_full * CHUNK
  row0 = sid * RPS

  @pl.loop(0, n_full)
  def _(k):
    pltpu.sync_copy(z, acc_s.at[pl.ds(row0 + k * CHUNK, CHUNK)])
  pltpu.sync_copy(z.at[pl.ds(0, rem)],
                  acc_s.at[pl.ds(row0 + n_full * CHUNK, rem)])

  @pl.when(sid == NS - 1)
  def _():
    pltpu.sync_copy(z.at[pl.ds(0, TAIL)], acc_s.at[pl.ds(NS * RPS, TAIL)])

  plsc.subcore_barrier()

  base = wid * EPW

  @pl.loop(0, N2)
  def _(k2):
    for b in range(NBUF):
      @pl.when(k2 > 0)
      def _():
        pltpu.make_async_copy(rows_v.at[b], acc_s.at[dst_v.at[b]],
                              sems[b]).wait()
      off = base + (k2 * NBUF + b) * CHUNK
      pltpu.sync_copy(src_hbm.at[pl.ds(off, CHUNK)], src_v.at[b])
      pltpu.sync_copy(dst_hbm.at[pl.ds(off, CHUNK)], dst_v.at[b])
      pltpu.sync_copy(x_hbm.at[src_v.at[b]], rows_v.at[b])
      pltpu.async_copy(rows_v.at[b], acc_s.at[dst_v.at[b]], sems[b],
                       add=True)

  for b in range(NBUF):
    pltpu.make_async_copy(rows_v.at[b], acc_s.at[dst_v.at[b]], sems[b]).wait()

  for t in range(N2 * NBUF, NCHUNK):
    off = base + t * CHUNK
    pltpu.sync_copy(src_hbm.at[pl.ds(off, CHUNK)], src_v.at[0])
    pltpu.sync_copy(dst_hbm.at[pl.ds(off, CHUNK)], dst_v.at[0])
    pltpu.sync_copy(x_hbm.at[src_v.at[0]], rows_v.at[0])
    pltpu.sync_copy(rows_v.at[0], acc_s.at[dst_v.at[0]], add=True)

  plsc.subcore_barrier()

  @pl.loop(0, n_full)
  def _(k):
    r = row0 + k * CHUNK
    pltpu.sync_copy(acc_s.at[pl.ds(r, CHUNK)], z)
    pltpu.sync_copy(z, acc_out.at[cid, pl.ds(r, CHUNK)])
  r = row0 + n_full * CHUNK
  pltpu.sync_copy(acc_s.at[pl.ds(r, rem)], z.at[pl.ds(0, rem)])
  pltpu.sync_copy(z.at[pl.ds(0, rem)], acc_out.at[cid, pl.ds(r, rem)])

  @pl.when(sid == NS - 1)
  def _():
    pltpu.sync_copy(acc_s.at[pl.ds(NS * RPS, TAIL)], z.at[pl.ds(0, TAIL)])
    pltpu.sync_copy(z.at[pl.ds(0, TAIL)],
                    acc_out.at[cid, pl.ds(NS * RPS, TAIL)])


_agg = pl.kernel(
    _agg_body,
    out_type=jax.ShapeDtypeStruct((NC, N, D), _F32),
    mesh=_MESH,
    scratch_types=[
        pltpu.VMEM((NBUF, CHUNK), jnp.int32),
        pltpu.VMEM((NBUF, CHUNK), jnp.int32),
        pltpu.VMEM((NBUF, CHUNK, D), _F32),
        pltpu.VMEM_SHARED((N, D), _F32),
    ] + [pltpu.SemaphoreType.DMA] * NBUF,
)


def _cnt_body(dst_hbm, cnt_out, dst_v, ones_v, cnt_s):
  cid = lax.axis_index("c")
  sid = lax.axis_index("s")
  wid = sid * NC + cid

  @pl.loop(0, CHUNK)
  def _(i):
    @pl.loop(0, D // 16)
    def _(j):
      ones_v[i, pl.ds(j * 16, 16)] = jnp.zeros((16,), _F32)

  n_full = RPS // CHUNK
  rem = RPS - n_full * CHUNK
  row0 = sid * RPS

  @pl.loop(0, n_full)
  def _(k):
    pltpu.sync_copy(ones_v, cnt_s.at[pl.ds(row0 + k * CHUNK, CHUNK)])
  pltpu.sync_copy(ones_v.at[pl.ds(0, rem)],
                  cnt_s.at[pl.ds(row0 + n_full * CHUNK, rem)])

  @pl.when(sid == NS - 1)
  def _():
    pltpu.sync_copy(ones_v.at[pl.ds(0, TAIL)], cnt_s.at[pl.ds(NS * RPS, TAIL)])

  @pl.loop(0, CHUNK)
  def _(i):
    @pl.loop(0, D // 16)
    def _(j):
      ones_v[i, pl.ds(j * 16, 16)] = jnp.full((16,), 1.0, _F32)

  plsc.subcore_barrier()

  base = wid * EPW

  @pl.loop(0, NCHUNK)
  def _(k):
    pltpu.sync_copy(dst_hbm.at[pl.ds(base + k * CHUNK, CHUNK)], dst_v)
    pltpu.sync_copy(ones_v, cnt_s.at[dst_v], add=True)

  plsc.subcore_barrier()

  @pl.loop(0, n_full)
  def _(k):
    r = row0 + k * CHUNK
    pltpu.sync_copy(cnt_s.at[pl.ds(r, CHUNK)], ones_v)
    pltpu.sync_copy(ones_v, cnt_out.at[cid, pl.ds(r, CHUNK)])
  r = row0 + n_full * CHUNK
  pltpu.sync_copy(cnt_s.at[pl.ds(r, rem)], ones_v.at[pl.ds(0, rem)])
  pltpu.sync_copy(ones_v.at[pl.ds(0, rem)], cnt_out.at[cid, pl.ds(r, rem)])

  @pl.when(sid == NS - 1)
  def _():
    pltpu.sync_copy(cnt_s.at[pl.ds(NS * RPS, TAIL)], ones_v.at[pl.ds(0, TAIL)])
    pltpu.sync_copy(ones_v.at[pl.ds(0, TAIL)],
                    cnt_out.at[cid, pl.ds(NS * RPS, TAIL)])


_counts = pl.kernel(
    _cnt_body,
    out_type=jax.ShapeDtypeStruct((NC, N, D), _F32),
    mesh=_MESH,
    scratch_types=[
        pltpu.VMEM((CHUNK,), jnp.int32),
        pltpu.VMEM((CHUNK, D), _F32),
        pltpu.VMEM_SHARED((N, D), _F32),
    ],
)


def _dot_t(a, w):
  return lax.dot_general(a, w, (((1,), (1,)), ((), ())),
                         precision=lax.Precision.HIGHEST,
                         preferred_element_type=_F32)


_BR = 1000


def _layer_tc(acc, rec, h, Wl, bl, Wr, relu):
  def body(acc_ref, rec_ref, h_ref, wl_ref, bl_ref, wr_ref, out_ref):
    mean = (acc_ref[0] + acc_ref[1]) * rec_ref[...]
    y = _dot_t(mean, wl_ref[...]) + _dot_t(h_ref[...], wr_ref[...]) + bl_ref[...]
    out_ref[...] = jnp.maximum(y, 0.0) if relu else y

  return pl.pallas_call(
      body,
      grid=(N // _BR,),
      in_specs=[
          pl.BlockSpec((NC, _BR, D), lambda i: (0, i, 0)),
          pl.BlockSpec((_BR, 1), lambda i: (i, 0)),
          pl.BlockSpec((_BR, D), lambda i: (i, 0)),
          pl.BlockSpec((D, D), lambda i: (0, 0)),
          pl.BlockSpec((1, D), lambda i: (0, 0)),
          pl.BlockSpec((D, D), lambda i: (0, 0)),
      ],
      out_specs=pl.BlockSpec((_BR, D), lambda i: (i, 0)),
      out_shape=jax.ShapeDtypeStruct((N, D), _F32),
  )(acc, rec, h, Wl, bl, Wr)


def _recip_tc(cnt):
  def body(cnt_ref, out_ref):
    c = cnt_ref[0, :, 0:1] + cnt_ref[1, :, 0:1]
    out_ref[...] = 1.0 / jnp.maximum(c, 1.0)

  return pl.pallas_call(
      body,
      grid=(N // _BR,),
      in_specs=[pl.BlockSpec((NC, _BR, D), lambda i: (0, i, 0))],
      out_specs=pl.BlockSpec((_BR, 1), lambda i: (i, 0)),
      out_shape=jax.ShapeDtypeStruct((N, 1), _F32),
  )(cnt)


def _last_layer_tc(acc, rec, h, Wl, bl, Wr, Wp, bp):
  g = N // _BR

  def body(acc_ref, rec_ref, h_ref, wl_ref, bl_ref, wr_ref, wp_ref, bp_ref,
           h3_ref, proj_ref, stats_ref, ssum, ssq):
    i = pl.program_id(0)
    mean = (acc_ref[0] + acc_ref[1]) * rec_ref[...]
    y = _dot_t(mean, wl_ref[...]) + _dot_t(h_ref[...], wr_ref[...]) + bl_ref[...]
    h3_ref[...] = y
    p = _dot_t(y, wp_ref[...]) + bp_ref[...]
    proj_ref[...] = p

    @pl.when(i == 0)
    def _():
      ssum[...] = jnp.zeros_like(ssum)
      ssq[...] = jnp.zeros_like(ssq)

    ssum[...] += jnp.sum(p, axis=0, keepdims=True)
    ssq[...] += jnp.sum(p * p, axis=0, keepdims=True)

    @pl.when(i == g - 1)
    def _():
      stats_ref[0:1, :] = ssum[...]
      stats_ref[1:2, :] = ssq[...]

  return pl.pallas_call(
      body,
      grid=(g,),
      in_specs=[
          pl.BlockSpec((NC, _BR, D), lambda i: (0, i, 0)),
          pl.BlockSpec((_BR, 1), lambda i: (i, 0)),
          pl.BlockSpec((_BR, D), lambda i: (i, 0)),
          pl.BlockSpec((D, D), lambda i: (0, 0)),
          pl.BlockSpec((1, D), lambda i: (0, 0)),
          pl.BlockSpec((D, D), lambda i: (0, 0)),
          pl.BlockSpec((D, D), lambda i: (0, 0)),
          pl.BlockSpec((1, D), lambda i: (0, 0)),
      ],
      out_specs=[
          pl.BlockSpec((_BR, D), lambda i: (i, 0)),
          pl.BlockSpec((_BR, D), lambda i: (i, 0)),
          pl.BlockSpec((2, D), lambda i: (0, 0)),
      ],
      out_shape=[
          jax.ShapeDtypeStruct((N, D), _F32),
          jax.ShapeDtypeStruct((N, D), _F32),
          jax.ShapeDtypeStruct((2, D), _F32),
      ],
      scratch_shapes=[pltpu.VMEM((1, D), _F32), pltpu.VMEM((1, D), _F32)],
  )(acc, rec, h, Wl, bl, Wr, Wp, bp)


def _bnorm_tc(proj, stats, gamma, beta):
  def body(proj_ref, stats_ref, g_ref, b_ref, out_ref):
    inv_n = 1.0 / N
    mu = stats_ref[0:1, :] * inv_n
    var = stats_ref[1:2, :] * inv_n - mu * mu
    scale = g_ref[...] * lax.rsqrt(var + 1e-5)
    out_ref[...] = proj_ref[...] * scale + (b_ref[...] - mu * scale)

  return pl.pallas_call(
      body,
      grid=(N // _BR,),
      in_specs=[
          pl.BlockSpec((_BR, D), lambda i: (i, 0)),
          pl.BlockSpec((2, D), lambda i: (0, 0)),
          pl.BlockSpec((1, D), lambda i: (0, 0)),
          pl.BlockSpec((1, D), lambda i: (0, 0)),
      ],
      out_specs=pl.BlockSpec((_BR, D), lambda i: (i, 0)),
      out_shape=jax.ShapeDtypeStruct((N, D), _F32),
  )(proj, stats, gamma, beta)


def kernel(x, edge_index, Wl0, bl0, Wr0, Wl1, bl1, Wr1, Wl2, bl2, Wr2,
           Wp, bp, gamma, beta):
  src = edge_index[0]
  dst = edge_index[1]
  bl0r = bl0.reshape(1, D)
  bl1r = bl1.reshape(1, D)
  bl2r = bl2.reshape(1, D)
  bpr = bp.reshape(1, D)
  gr = gamma.reshape(1, D)
  br = beta.reshape(1, D)

  cnt = _counts(dst)
  rec = _recip_tc(cnt)
  acc0 = _agg(x, src, dst)
  h1 = _layer_tc(acc0, rec, x, Wl0, bl0r, Wr0, True)
  acc1 = _agg(h1, src, dst)
  h2 = _layer_tc(acc1, rec, h1, Wl1, bl1r, Wr1, True)
  acc2 = _agg(h2, src, dst)
  h3, proj, stats = _last_layer_tc(acc2, rec, h2, Wl2, bl2r, Wr2, Wp, bpr)
  h_out = _bnorm_tc(proj, stats, gr, br)
  return (h3, h_out)


# speedup vs baseline: 7.8000x; 1.4763x over previous
import jax
import jax.numpy as jnp
from jax import lax
from jax.experimental import pallas as pl
from jax.experimental.pallas import tpu as pltpu
from jax.experimental.pallas import tpu_sc as plsc

N = 10000
E = 320000
D = 128

NC = 2
NS = 16
NW = NC * NS
EPW = E // NW
CHUNK = 80
NCHUNK = EPW // CHUNK
RPS = 624
TAIL = N - NS * RPS

_F32 = jnp.float32
_MESH = plsc.VectorSubcoreMesh(core_axis_name="c", subcore_axis_name="s")


NBUF = 4
N2 = NCHUNK // NBUF


def _agg_body(x_hbm, src_hbm, dst_hbm, acc_out, src_v, dst_v, rows_v, acc_s,
              *sems):
  ssem = sems[:NBUF]
  gsem = sems[NBUF:]
  cid = lax.axis_index("c")
  sid = lax.axis_index("s")
  wid = sid * NC + cid
  z = rows_v.at[0]

  @pl.loop(0, CHUNK)
  def _(i):
    @pl.loop(0, D // 16)
    def _(j):
      rows_v[0, i, pl.ds(j * 16, 16)] = jnp.zeros((16,), _F32)

  n_full = RPS // CHUNK
  rem = RPS - n_full * CHUNK
  row0 = sid * RPS

  @pl.loop(0, n_full)
  def _(k):
    pltpu.sync_copy(z, acc_s.at[pl.ds(row0 + k * CHUNK, CHUNK)])
  pltpu.sync_copy(z.at[pl.ds(0, rem)],
                  acc_s.at[pl.ds(row0 + n_full * CHUNK, rem)])

  @pl.when(sid == NS - 1)
  def _():
    pltpu.sync_copy(z.at[pl.ds(0, TAIL)], acc_s.at[pl.ds(NS * RPS, TAIL)])

  plsc.subcore_barrier()

  base = wid * EPW

  def wait_scatter(b):
    pltpu.make_async_copy(rows_v.at[b], acc_s.at[dst_v.at[b]],
                          ssem[b]).wait()

  def wait_gather_then_scatter(b):
    pltpu.make_async_copy(x_hbm.at[src_v.at[b]], rows_v.at[b],
                          gsem[b]).wait()
    pltpu.async_copy(rows_v.at[b], acc_s.at[dst_v.at[b]], ssem[b], add=True)

  @pl.loop(0, N2)
  def _(k2):
    for b in range(NBUF):
      @pl.when(k2 > 0)
      def _():
        wait_scatter(b)
      off = base + (k2 * NBUF + b) * CHUNK
      pltpu.sync_copy(src_hbm.at[pl.ds(off, CHUNK)], src_v.at[b])
      pltpu.sync_copy(dst_hbm.at[pl.ds(off, CHUNK)], dst_v.at[b])
      pltpu.async_copy(x_hbm.at[src_v.at[b]], rows_v.at[b], gsem[b])
      pb = (b - 1) % NBUF
      if b == 0:
        @pl.when(k2 > 0)
        def _():
          wait_gather_then_scatter(pb)
      else:
        wait_gather_then_scatter(pb)

  wait_gather_then_scatter(NBUF - 1)
  for b in range(NBUF):
    wait_scatter(b)

  for t in range(N2 * NBUF, NCHUNK):
    off = base + t * CHUNK
    pltpu.sync_copy(src_hbm.at[pl.ds(off, CHUNK)], src_v.at[0])
    pltpu.sync_copy(dst_hbm.at[pl.ds(off, CHUNK)], dst_v.at[0])
    pltpu.sync_copy(x_hbm.at[src_v.at[0]], rows_v.at[0])
    pltpu.sync_copy(rows_v.at[0], acc_s.at[dst_v.at[0]], add=True)

  plsc.subcore_barrier()

  @pl.loop(0, n_full)
  def _(k):
    r = row0 + k * CHUNK
    pltpu.sync_copy(acc_s.at[pl.ds(r, CHUNK)], z)
    pltpu.sync_copy(z, acc_out.at[cid, pl.ds(r, CHUNK)])
  r = row0 + n_full * CHUNK
  pltpu.sync_copy(acc_s.at[pl.ds(r, rem)], z.at[pl.ds(0, rem)])
  pltpu.sync_copy(z.at[pl.ds(0, rem)], acc_out.at[cid, pl.ds(r, rem)])

  @pl.when(sid == NS - 1)
  def _():
    pltpu.sync_copy(acc_s.at[pl.ds(NS * RPS, TAIL)], z.at[pl.ds(0, TAIL)])
    pltpu.sync_copy(z.at[pl.ds(0, TAIL)],
                    acc_out.at[cid, pl.ds(NS * RPS, TAIL)])


_agg = pl.kernel(
    _agg_body,
    out_type=jax.ShapeDtypeStruct((NC, N, D), _F32),
    mesh=_MESH,
    scratch_types=[
        pltpu.VMEM((NBUF, CHUNK), jnp.int32),
        pltpu.VMEM((NBUF, CHUNK), jnp.int32),
        pltpu.VMEM((NBUF, CHUNK, D), _F32),
        pltpu.VMEM_SHARED((N, D), _F32),
    ] + [pltpu.SemaphoreType.DMA] * (2 * NBUF),
)


def _cnt_body(dst_hbm, cnt_out, dst_v, ones_v, cnt_s):
  cid = lax.axis_index("c")
  sid = lax.axis_index("s")
  wid = sid * NC + cid

  @pl.loop(0, CHUNK)
  def _(i):
    @pl.loop(0, D // 16)
    def _(j):
      ones_v[i, pl.ds(j * 16, 16)] = jnp.zeros((16,), _F32)

  n_full = RPS // CHUNK
  rem = RPS - n_full * CHUNK
  row0 = sid * RPS

  @pl.loop(0, n_full)
  def _(k):
    pltpu.sync_copy(ones_v, cnt_s.at[pl.ds(row0 + k * CHUNK, CHUNK)])
  pltpu.sync_copy(ones_v.at[pl.ds(0, rem)],
                  cnt_s.at[pl.ds(row0 + n_full * CHUNK, rem)])

  @pl.when(sid == NS - 1)
  def _():
    pltpu.sync_copy(ones_v.at[pl.ds(0, TAIL)], cnt_s.at[pl.ds(NS * RPS, TAIL)])

  @pl.loop(0, CHUNK)
  def _(i):
    @pl.loop(0, D // 16)
    def _(j):
      ones_v[i, pl.ds(j * 16, 16)] = jnp.full((16,), 1.0, _F32)

  plsc.subcore_barrier()

  base = wid * EPW

  @pl.loop(0, NCHUNK)
  def _(k):
    pltpu.sync_copy(dst_hbm.at[pl.ds(base + k * CHUNK, CHUNK)], dst_v)
    pltpu.sync_copy(ones_v, cnt_s.at[dst_v], add=True)

  plsc.subcore_barrier()

  @pl.loop(0, n_full)
  def _(k):
    r = row0 + k * CHUNK
    pltpu.sync_copy(cnt_s.at[pl.ds(r, CHUNK)], ones_v)
    pltpu.sync_copy(ones_v, cnt_out.at[cid, pl.ds(r, CHUNK)])
  r = row0 + n_full * CHUNK
  pltpu.sync_copy(cnt_s.at[pl.ds(r, rem)], ones_v.at[pl.ds(0, rem)])
  pltpu.sync_copy(ones_v.at[pl.ds(0, rem)], cnt_out.at[cid, pl.ds(r, rem)])

  @pl.when(sid == NS - 1)
  def _():
    pltpu.sync_copy(cnt_s.at[pl.ds(NS * RPS, TAIL)], ones_v.at[pl.ds(0, TAIL)])
    pltpu.sync_copy(ones_v.at[pl.ds(0, TAIL)],
                    cnt_out.at[cid, pl.ds(NS * RPS, TAIL)])


_counts = pl.kernel(
    _cnt_body,
    out_type=jax.ShapeDtypeStruct((NC, N, D), _F32),
    mesh=_MESH,
    scratch_types=[
        pltpu.VMEM((CHUNK,), jnp.int32),
        pltpu.VMEM((CHUNK, D), _F32),
        pltpu.VMEM_SHARED((N, D), _F32),
    ],
)


def _dot_t(a, w):
  return lax.dot_general(a, w, (((1,), (1,)), ((), ())),
                         precision=lax.Precision.HIGHEST,
                         preferred_element_type=_F32)


_BR = 1000


def _layer_tc(acc, rec, h, Wl, bl, Wr, relu):
  def body(acc_ref, rec_ref, h_ref, wl_ref, bl_ref, wr_ref, out_ref):
    mean = (acc_ref[0] + acc_ref[1]) * rec_ref[...]
    y = _dot_t(mean, wl_ref[...]) + _dot_t(h_ref[...], wr_ref[...]) + bl_ref[...]
    out_ref[...] = jnp.maximum(y, 0.0) if relu else y

  return pl.pallas_call(
      body,
      grid=(N // _BR,),
      in_specs=[
          pl.BlockSpec((NC, _BR, D), lambda i: (0, i, 0)),
          pl.BlockSpec((_BR, 1), lambda i: (i, 0)),
          pl.BlockSpec((_BR, D), lambda i: (i, 0)),
          pl.BlockSpec((D, D), lambda i: (0, 0)),
          pl.BlockSpec((1, D), lambda i: (0, 0)),
          pl.BlockSpec((D, D), lambda i: (0, 0)),
      ],
      out_specs=pl.BlockSpec((_BR, D), lambda i: (i, 0)),
      out_shape=jax.ShapeDtypeStruct((N, D), _F32),
  )(acc, rec, h, Wl, bl, Wr)


def _recip_tc(cnt):
  def body(cnt_ref, out_ref):
    c = cnt_ref[0, :, 0:1] + cnt_ref[1, :, 0:1]
    out_ref[...] = 1.0 / jnp.maximum(c, 1.0)

  return pl.pallas_call(
      body,
      grid=(N // _BR,),
      in_specs=[pl.BlockSpec((NC, _BR, D), lambda i: (0, i, 0))],
      out_specs=pl.BlockSpec((_BR, 1), lambda i: (i, 0)),
      out_shape=jax.ShapeDtypeStruct((N, 1), _F32),
  )(cnt)


def _last_layer_tc(acc, rec, h, Wl, bl, Wr, Wp, bp):
  g = N // _BR

  def body(acc_ref, rec_ref, h_ref, wl_ref, bl_ref, wr_ref, wp_ref, bp_ref,
           h3_ref, proj_ref, stats_ref, ssum, ssq):
    i = pl.program_id(0)
    mean = (acc_ref[0] + acc_ref[1]) * rec_ref[...]
    y = _dot_t(mean, wl_ref[...]) + _dot_t(h_ref[...], wr_ref[...]) + bl_ref[...]
    h3_ref[...] = y
    p = _dot_t(y, wp_ref[...]) + bp_ref[...]
    proj_ref[...] = p

    @pl.when(i == 0)
    def _():
      ssum[...] = jnp.zeros_like(ssum)
      ssq[...] = jnp.zeros_like(ssq)

    ssum[...] += jnp.sum(p, axis=0, keepdims=True)
    ssq[...] += jnp.sum(p * p, axis=0, keepdims=True)

    @pl.when(i == g - 1)
    def _():
      stats_ref[0:1, :] = ssum[...]
      stats_ref[1:2, :] = ssq[...]

  return pl.pallas_call(
      body,
      grid=(g,),
      in_specs=[
          pl.BlockSpec((NC, _BR, D), lambda i: (0, i, 0)),
          pl.BlockSpec((_BR, 1), lambda i: (i, 0)),
          pl.BlockSpec((_BR, D), lambda i: (i, 0)),
          pl.BlockSpec((D, D), lambda i: (0, 0)),
          pl.BlockSpec((1, D), lambda i: (0, 0)),
          pl.BlockSpec((D, D), lambda i: (0, 0)),
          pl.BlockSpec((D, D), lambda i: (0, 0)),
          pl.BlockSpec((1, D), lambda i: (0, 0)),
      ],
      out_specs=[
          pl.BlockSpec((_BR, D), lambda i: (i, 0)),
          pl.BlockSpec((_BR, D), lambda i: (i, 0)),
          pl.BlockSpec((2, D), lambda i: (0, 0)),
      ],
      out_shape=[
          jax.ShapeDtypeStruct((N, D), _F32),
          jax.ShapeDtypeStruct((N, D), _F32),
          jax.ShapeDtypeStruct((2, D), _F32),
      ],
      scratch_shapes=[pltpu.VMEM((1, D), _F32), pltpu.VMEM((1, D), _F32)],
  )(acc, rec, h, Wl, bl, Wr, Wp, bp)


def _bnorm_tc(proj, stats, gamma, beta):
  def body(proj_ref, stats_ref, g_ref, b_ref, out_ref):
    inv_n = 1.0 / N
    mu = stats_ref[0:1, :] * inv_n
    var = stats_ref[1:2, :] * inv_n - mu * mu
    scale = g_ref[...] * lax.rsqrt(var + 1e-5)
    out_ref[...] = proj_ref[...] * scale + (b_ref[...] - mu * scale)

  return pl.pallas_call(
      body,
      grid=(N // _BR,),
      in_specs=[
          pl.BlockSpec((_BR, D), lambda i: (i, 0)),
          pl.BlockSpec((2, D), lambda i: (0, 0)),
          pl.BlockSpec((1, D), lambda i: (0, 0)),
          pl.BlockSpec((1, D), lambda i: (0, 0)),
      ],
      out_specs=pl.BlockSpec((_BR, D), lambda i: (i, 0)),
      out_shape=jax.ShapeDtypeStruct((N, D), _F32),
  )(proj, stats, gamma, beta)


def kernel(x, edge_index, Wl0, bl0, Wr0, Wl1, bl1, Wr1, Wl2, bl2, Wr2,
           Wp, bp, gamma, beta):
  src = edge_index[0]
  dst = edge_index[1]
  bl0r = bl0.reshape(1, D)
  bl1r = bl1.reshape(1, D)
  bl2r = bl2.reshape(1, D)
  bpr = bp.reshape(1, D)
  gr = gamma.reshape(1, D)
  br = beta.reshape(1, D)

  cnt = _counts(dst)
  rec = _recip_tc(cnt)
  acc0 = _agg(x, src, dst)
  h1 = _layer_tc(acc0, rec, x, Wl0, bl0r, Wr0, True)
  acc1 = _agg(h1, src, dst)
  h2 = _layer_tc(acc1, rec, h1, Wl1, bl1r, Wr1, True)
  acc2 = _agg(h2, src, dst)
  h3, proj, stats = _last_layer_tc(acc2, rec, h2, Wl2, bl2r, Wr2, Wp, bpr)
  h_out = _bnorm_tc(proj, stats, gr, br)
  return (h3, h_out)


# speedup vs baseline: 8.4194x; 1.0794x over previous
import jax
import jax.numpy as jnp
from jax import lax
from jax.experimental import pallas as pl
from jax.experimental.pallas import tpu as pltpu
from jax.experimental.pallas import tpu_sc as plsc

N = 10000
E = 320000
D = 128

NC = 2
NS = 16
NW = NC * NS
EPW = E // NW
CHUNK = 80
NCHUNK = EPW // CHUNK
RPS = 624
TAIL = N - NS * RPS

_F32 = jnp.float32
_MESH = plsc.VectorSubcoreMesh(core_axis_name="c", subcore_axis_name="s")


NBUF = 4
N2 = NCHUNK // NBUF


def _agg_body(x_hbm, src_hbm, dst_hbm, acc_out, src_v, dst_v, rows_v, acc_s,
              *sems):
  ssem = sems[:NBUF]
  gsem = sems[NBUF:2 * NBUF]
  isem = sems[2 * NBUF:]
  cid = lax.axis_index("c")
  sid = lax.axis_index("s")
  wid = sid * NC + cid
  z = rows_v.at[0]

  @pl.loop(0, CHUNK)
  def _(i):
    @pl.loop(0, D // 16)
    def _(j):
      rows_v[0, i, pl.ds(j * 16, 16)] = jnp.zeros((16,), _F32)

  n_full = RPS // CHUNK
  rem = RPS - n_full * CHUNK
  row0 = sid * RPS

  @pl.loop(0, n_full)
  def _(k):
    pltpu.sync_copy(z, acc_s.at[pl.ds(row0 + k * CHUNK, CHUNK)])
  pltpu.sync_copy(z.at[pl.ds(0, rem)],
                  acc_s.at[pl.ds(row0 + n_full * CHUNK, rem)])

  @pl.when(sid == NS - 1)
  def _():
    pltpu.sync_copy(z.at[pl.ds(0, TAIL)], acc_s.at[pl.ds(NS * RPS, TAIL)])

  plsc.subcore_barrier()

  base = wid * EPW

  def wait_scatter(b):
    pltpu.make_async_copy(rows_v.at[b], acc_s.at[dst_v.at[b]],
                          ssem[b]).wait()

  def wait_gather_then_scatter(b):
    pltpu.make_async_copy(x_hbm.at[src_v.at[b]], rows_v.at[b],
                          gsem[b]).wait()
    pltpu.async_copy(rows_v.at[b], acc_s.at[dst_v.at[b]], ssem[b], add=True)

  @pl.loop(0, N2)
  def _(k2):
    for b in range(NBUF):
      @pl.when(k2 > 0)
      def _():
        wait_scatter(b)
      off = base + (k2 * NBUF + b) * CHUNK
      pltpu.sync_copy(src_hbm.at[pl.ds(off, CHUNK)], src_v.at[b])
      pltpu.sync_copy(dst_hbm.at[pl.ds(off, CHUNK)], dst_v.at[b])
      pltpu.async_copy(x_hbm.at[src_v.at[b]], rows_v.at[b], gsem[b])
      pb = (b - 1) % NBUF
      if b == 0:
        @pl.when(k2 > 0)
        def _():
          wait_gather_then_scatter(pb)
      else:
        wait_gather_then_scatter(pb)

  wait_gather_then_scatter(NBUF - 1)
  for b in range(NBUF):
    wait_scatter(b)

  for t in range(N2 * NBUF, NCHUNK):
    off = base + t * CHUNK
    pltpu.sync_copy(src_hbm.at[pl.ds(off, CHUNK)], src_v.at[0])
    pltpu.sync_copy(dst_hbm.at[pl.ds(off, CHUNK)], dst_v.at[0])
    pltpu.sync_copy(x_hbm.at[src_v.at[0]], rows_v.at[0])
    pltpu.sync_copy(rows_v.at[0], acc_s.at[dst_v.at[0]], add=True)

  plsc.subcore_barrier()

  @pl.loop(0, n_full)
  def _(k):
    r = row0 + k * CHUNK
    pltpu.sync_copy(acc_s.at[pl.ds(r, CHUNK)], z)
    pltpu.sync_copy(z, acc_out.at[cid, pl.ds(r, CHUNK)])
  r = row0 + n_full * CHUNK
  pltpu.sync_copy(acc_s.at[pl.ds(r, rem)], z.at[pl.ds(0, rem)])
  pltpu.sync_copy(z.at[pl.ds(0, rem)], acc_out.at[cid, pl.ds(r, rem)])

  @pl.when(sid == NS - 1)
  def _():
    pltpu.sync_copy(acc_s.at[pl.ds(NS * RPS, TAIL)], z.at[pl.ds(0, TAIL)])
    pltpu.sync_copy(z.at[pl.ds(0, TAIL)],
                    acc_out.at[cid, pl.ds(NS * RPS, TAIL)])


_agg = pl.kernel(
    _agg_body,
    out_type=jax.ShapeDtypeStruct((NC, N, D), _F32),
    mesh=_MESH,
    scratch_types=[
        pltpu.VMEM((NBUF, CHUNK), jnp.int32),
        pltpu.VMEM((NBUF, CHUNK), jnp.int32),
        pltpu.VMEM((NBUF, CHUNK, D), _F32),
        pltpu.VMEM_SHARED((N, D), _F32),
    ] + [pltpu.SemaphoreType.DMA] * (2 * NBUF),
)


CNBUF = 8
C2 = NCHUNK // CNBUF


def _cnt_body(dst_hbm, cnt_out, dst_v, ones_v, cnt_s, *sems):
  cid = lax.axis_index("c")
  sid = lax.axis_index("s")
  wid = sid * NC + cid

  @pl.loop(0, CHUNK)
  def _(i):
    @pl.loop(0, D // 16)
    def _(j):
      ones_v[i, pl.ds(j * 16, 16)] = jnp.zeros((16,), _F32)

  n_full = RPS // CHUNK
  rem = RPS - n_full * CHUNK
  row0 = sid * RPS

  @pl.loop(0, n_full)
  def _(k):
    pltpu.sync_copy(ones_v, cnt_s.at[pl.ds(row0 + k * CHUNK, CHUNK)])
  pltpu.sync_copy(ones_v.at[pl.ds(0, rem)],
                  cnt_s.at[pl.ds(row0 + n_full * CHUNK, rem)])

  @pl.when(sid == NS - 1)
  def _():
    pltpu.sync_copy(ones_v.at[pl.ds(0, TAIL)], cnt_s.at[pl.ds(NS * RPS, TAIL)])

  @pl.loop(0, CHUNK)
  def _(i):
    @pl.loop(0, D // 16)
    def _(j):
      ones_v[i, pl.ds(j * 16, 16)] = jnp.full((16,), 1.0, _F32)

  plsc.subcore_barrier()

  base = wid * EPW

  @pl.loop(0, C2)
  def _(k2):
    for b in range(CNBUF):
      @pl.when(k2 > 0)
      def _():
        pltpu.make_async_copy(ones_v, cnt_s.at[dst_v.at[b]], sems[b]).wait()
      off = base + (k2 * CNBUF + b) * CHUNK
      pltpu.sync_copy(dst_hbm.at[pl.ds(off, CHUNK)], dst_v.at[b])
      pltpu.async_copy(ones_v, cnt_s.at[dst_v.at[b]], sems[b], add=True)

  for b in range(CNBUF):
    pltpu.make_async_copy(ones_v, cnt_s.at[dst_v.at[b]], sems[b]).wait()

  for t in range(C2 * CNBUF, NCHUNK):
    off = base + t * CHUNK
    pltpu.sync_copy(dst_hbm.at[pl.ds(off, CHUNK)], dst_v.at[0])
    pltpu.sync_copy(ones_v, cnt_s.at[dst_v.at[0]], add=True)

  plsc.subcore_barrier()

  @pl.loop(0, n_full)
  def _(k):
    r = row0 + k * CHUNK
    pltpu.sync_copy(cnt_s.at[pl.ds(r, CHUNK)], ones_v)
    pltpu.sync_copy(ones_v, cnt_out.at[cid, pl.ds(r, CHUNK)])
  r = row0 + n_full * CHUNK
  pltpu.sync_copy(cnt_s.at[pl.ds(r, rem)], ones_v.at[pl.ds(0, rem)])
  pltpu.sync_copy(ones_v.at[pl.ds(0, rem)], cnt_out.at[cid, pl.ds(r, rem)])

  @pl.when(sid == NS - 1)
  def _():
    pltpu.sync_copy(cnt_s.at[pl.ds(NS * RPS, TAIL)], ones_v.at[pl.ds(0, TAIL)])
    pltpu.sync_copy(ones_v.at[pl.ds(0, TAIL)],
                    cnt_out.at[cid, pl.ds(NS * RPS, TAIL)])


_counts = pl.kernel(
    _cnt_body,
    out_type=jax.ShapeDtypeStruct((NC, N, D), _F32),
    mesh=_MESH,
    scratch_types=[
        pltpu.VMEM((CNBUF, CHUNK), jnp.int32),
        pltpu.VMEM((CHUNK, D), _F32),
        pltpu.VMEM_SHARED((N, D), _F32),
    ] + [pltpu.SemaphoreType.DMA] * CNBUF,
)


def _dot_t(a, w):
  return lax.dot_general(a, w, (((1,), (1,)), ((), ())),
                         precision=lax.Precision.HIGHEST,
                         preferred_element_type=_F32)


_BR = 1000


def _layer_tc(acc, rec, h, Wl, bl, Wr, relu):
  def body(acc_ref, rec_ref, h_ref, wl_ref, bl_ref, wr_ref, out_ref):
    mean = (acc_ref[0] + acc_ref[1]) * rec_ref[...]
    y = _dot_t(mean, wl_ref[...]) + _dot_t(h_ref[...], wr_ref[...]) + bl_ref[...]
    out_ref[...] = jnp.maximum(y, 0.0) if relu else y

  return pl.pallas_call(
      body,
      grid=(N // _BR,),
      in_specs=[
          pl.BlockSpec((NC, _BR, D), lambda i: (0, i, 0)),
          pl.BlockSpec((_BR, 1), lambda i: (i, 0)),
          pl.BlockSpec((_BR, D), lambda i: (i, 0)),
          pl.BlockSpec((D, D), lambda i: (0, 0)),
          pl.BlockSpec((1, D), lambda i: (0, 0)),
          pl.BlockSpec((D, D), lambda i: (0, 0)),
      ],
      out_specs=pl.BlockSpec((_BR, D), lambda i: (i, 0)),
      out_shape=jax.ShapeDtypeStruct((N, D), _F32),
  )(acc, rec, h, Wl, bl, Wr)


def _recip_tc(cnt):
  def body(cnt_ref, out_ref):
    c = cnt_ref[0, :, 0:1] + cnt_ref[1, :, 0:1]
    out_ref[...] = 1.0 / jnp.maximum(c, 1.0)

  return pl.pallas_call(
      body,
      grid=(N // _BR,),
      in_specs=[pl.BlockSpec((NC, _BR, D), lambda i: (0, i, 0))],
      out_specs=pl.BlockSpec((_BR, 1), lambda i: (i, 0)),
      out_shape=jax.ShapeDtypeStruct((N, 1), _F32),
  )(cnt)


def _last_layer_tc(acc, rec, h, Wl, bl, Wr, Wp, bp):
  g = N // _BR

  def body(acc_ref, rec_ref, h_ref, wl_ref, bl_ref, wr_ref, wp_ref, bp_ref,
           h3_ref, proj_ref, stats_ref, ssum, ssq):
    i = pl.program_id(0)
    mean = (acc_ref[0] + acc_ref[1]) * rec_ref[...]
    y = _dot_t(mean, wl_ref[...]) + _dot_t(h_ref[...], wr_ref[...]) + bl_ref[...]
    h3_ref[...] = y
    p = _dot_t(y, wp_ref[...]) + bp_ref[...]
    proj_ref[...] = p

    @pl.when(i == 0)
    def _():
      ssum[...] = jnp.zeros_like(ssum)
      ssq[...] = jnp.zeros_like(ssq)

    ssum[...] += jnp.sum(p, axis=0, keepdims=True)
    ssq[...] += jnp.sum(p * p, axis=0, keepdims=True)

    @pl.when(i == g - 1)
    def _():
      stats_ref[0:1, :] = ssum[...]
      stats_ref[1:2, :] = ssq[...]

  return pl.pallas_call(
      body,
      grid=(g,),
      in_specs=[
          pl.BlockSpec((NC, _BR, D), lambda i: (0, i, 0)),
          pl.BlockSpec((_BR, 1), lambda i: (i, 0)),
          pl.BlockSpec((_BR, D), lambda i: (i, 0)),
          pl.BlockSpec((D, D), lambda i: (0, 0)),
          pl.BlockSpec((1, D), lambda i: (0, 0)),
          pl.BlockSpec((D, D), lambda i: (0, 0)),
          pl.BlockSpec((D, D), lambda i: (0, 0)),
          pl.BlockSpec((1, D), lambda i: (0, 0)),
      ],
      out_specs=[
          pl.BlockSpec((_BR, D), lambda i: (i, 0)),
          pl.BlockSpec((_BR, D), lambda i: (i, 0)),
          pl.BlockSpec((2, D), lambda i: (0, 0)),
      ],
      out_shape=[
          jax.ShapeDtypeStruct((N, D), _F32),
          jax.ShapeDtypeStruct((N, D), _F32),
          jax.ShapeDtypeStruct((2, D), _F32),
      ],
      scratch_shapes=[pltpu.VMEM((1, D), _F32), pltpu.VMEM((1, D), _F32)],
  )(acc, rec, h, Wl, bl, Wr, Wp, bp)


def _bnorm_tc(proj, stats, gamma, beta):
  def body(proj_ref, stats_ref, g_ref, b_ref, out_ref):
    inv_n = 1.0 / N
    mu = stats_ref[0:1, :] * inv_n
    var = stats_ref[1:2, :] * inv_n - mu * mu
    scale = g_ref[...] * lax.rsqrt(var + 1e-5)
    out_ref[...] = proj_ref[...] * scale + (b_ref[...] - mu * scale)

  return pl.pallas_call(
      body,
      grid=(N // _BR,),
      in_specs=[
          pl.BlockSpec((_BR, D), lambda i: (i, 0)),
          pl.BlockSpec((2, D), lambda i: (0, 0)),
          pl.BlockSpec((1, D), lambda i: (0, 0)),
          pl.BlockSpec((1, D), lambda i: (0, 0)),
      ],
      out_specs=pl.BlockSpec((_BR, D), lambda i: (i, 0)),
      out_shape=jax.ShapeDtypeStruct((N, D), _F32),
  )(proj, stats, gamma, beta)


def kernel(x, edge_index, Wl0, bl0, Wr0, Wl1, bl1, Wr1, Wl2, bl2, Wr2,
           Wp, bp, gamma, beta):
  src = edge_index[0]
  dst = edge_index[1]
  bl0r = bl0.reshape(1, D)
  bl1r = bl1.reshape(1, D)
  bl2r = bl2.reshape(1, D)
  bpr = bp.reshape(1, D)
  gr = gamma.reshape(1, D)
  br = beta.reshape(1, D)

  cnt = _counts(dst)
  rec = _recip_tc(cnt)
  acc0 = _agg(x, src, dst)
  h1 = _layer_tc(acc0, rec, x, Wl0, bl0r, Wr0, True)
  acc1 = _agg(h1, src, dst)
  h2 = _layer_tc(acc1, rec, h1, Wl1, bl1r, Wr1, True)
  acc2 = _agg(h2, src, dst)
  h3, proj, stats = _last_layer_tc(acc2, rec, h2, Wl2, bl2r, Wr2, Wp, bpr)
  h_out = _bnorm_tc(proj, stats, gr, br)
  return (h3, h_out)


# speedup vs baseline: 9.8448x; 1.1693x over previous
import jax
import jax.numpy as jnp
from jax import lax
from jax.experimental import pallas as pl
from jax.experimental.pallas import tpu as pltpu
from jax.experimental.pallas import tpu_sc as plsc

N = 10000
E = 320000
D = 128

NC = 2
NS = 16
NW = NC * NS
EPW = E // NW
CHUNK = 80
NCHUNK = EPW // CHUNK
RPS = 624
TAIL = N - NS * RPS

_F32 = jnp.float32
_MESH = plsc.VectorSubcoreMesh(core_axis_name="c", subcore_axis_name="s")


NBUF = 4
N2 = NCHUNK // NBUF


BLKE = NBUF * CHUNK
N4 = N2 // 4


def _agg_body(x_hbm, src_hbm, dst_hbm, acc_out, src_i, dst_i, rows_v, acc_s,
              *sems):
  ssem = sems[:NBUF]
  gsem = sems[NBUF:2 * NBUF]
  s_isem = sems[2 * NBUF:2 * NBUF + 4]
  d_isem = sems[2 * NBUF + 4:2 * NBUF + 8]
  cid = lax.axis_index("c")
  sid = lax.axis_index("s")
  wid = sid * NC + cid
  z = rows_v.at[0]

  @pl.loop(0, CHUNK)
  def _(i):
    @pl.loop(0, D // 16)
    def _(j):
      rows_v[0, i, pl.ds(j * 16, 16)] = jnp.zeros((16,), _F32)

  n_full = RPS // CHUNK
  rem = RPS - n_full * CHUNK
  row0 = sid * RPS

  @pl.loop(0, n_full)
  def _(k):
    pltpu.sync_copy(z, acc_s.at[pl.ds(row0 + k * CHUNK, CHUNK)])
  pltpu.sync_copy(z.at[pl.ds(0, rem)],
                  acc_s.at[pl.ds(row0 + n_full * CHUNK, rem)])

  @pl.when(sid == NS - 1)
  def _():
    pltpu.sync_copy(z.at[pl.ds(0, TAIL)], acc_s.at[pl.ds(NS * RPS, TAIL)])

  plsc.subcore_barrier()

  base = wid * EPW

  def sidx(q, b):
    return src_i.at[q * NBUF + b]

  def didx(q, b):
    return dst_i.at[q * NBUF + b]

  def idx_load(k2v, q):
    for b in range(NBUF):
      off = base + (k2v * NBUF + b) * CHUNK
      pltpu.async_copy(src_hbm.at[pl.ds(off, CHUNK)], sidx(q, b), s_isem[q])
      pltpu.async_copy(dst_hbm.at[pl.ds(off, CHUNK)], didx(q, b), d_isem[q])

  def idx_wait(k2v, q):
    for b in range(NBUF):
      off = base + (k2v * NBUF + b) * CHUNK
      pltpu.make_async_copy(src_hbm.at[pl.ds(off, CHUNK)], sidx(q, b),
                            s_isem[q]).wait()
      pltpu.make_async_copy(dst_hbm.at[pl.ds(off, CHUNK)], didx(q, b),
                            d_isem[q]).wait()

  def wait_scatter(q, b):
    pltpu.make_async_copy(rows_v.at[b], acc_s.at[didx(q, b)], ssem[b]).wait()

  def wait_gather_then_scatter(q, b):
    pltpu.make_async_copy(x_hbm.at[sidx(q, b)], rows_v.at[b], gsem[b]).wait()
    pltpu.async_copy(rows_v.at[b], acc_s.at[didx(q, b)], ssem[b], add=True)

  def ring_cycle(k2v, q, first_pred):
    idx_wait(k2v, q)
    for b in range(NBUF):
      if first_pred is None:
        wait_scatter((q - 1) % 4, b)
      else:
        @pl.when(first_pred)
        def _():
          wait_scatter((q - 1) % 4, b)
      pltpu.async_copy(x_hbm.at[sidx(q, b)], rows_v.at[b], gsem[b])
      pb = (b - 1) % NBUF
      if b == 0:
        if first_pred is None:
          wait_gather_then_scatter((q - 1) % 4, pb)
        else:
          @pl.when(first_pred)
          def _():
            wait_gather_then_scatter((q - 1) % 4, pb)
      else:
        wait_gather_then_scatter(q, pb)

  idx_load(0, 0)
  idx_load(1, 1)

  @pl.loop(0, N4)
  def _(g):
    for c in range(4):
      k2v = g * 4 + c
      idx_load(k2v + 2, (c + 2) % 4)
      ring_cycle(k2v, c, (g > 0) if c == 0 else None)

  for k2v in range(N4 * 4, N2):
    q = k2v % 4
    if k2v + 2 <= N2 - 1:
      idx_load(k2v + 2, (q + 2) % 4)
    ring_cycle(k2v, q, None)

  qlast = (N2 - 1) % 4
  wait_gather_then_scatter(qlast, NBUF - 1)
  for b in range(NBUF):
    wait_scatter(qlast, b)

  for t in range(N2 * NBUF, NCHUNK):
    off = base + t * CHUNK
    pltpu.sync_copy(src_hbm.at[pl.ds(off, CHUNK)], sidx(0, 0))
    pltpu.sync_copy(dst_hbm.at[pl.ds(off, CHUNK)], didx(0, 0))
    pltpu.sync_copy(x_hbm.at[sidx(0, 0)], rows_v.at[0])
    pltpu.sync_copy(rows_v.at[0], acc_s.at[didx(0, 0)], add=True)

  plsc.subcore_barrier()

  @pl.loop(0, n_full)
  def _(k):
    r = row0 + k * CHUNK
    pltpu.sync_copy(acc_s.at[pl.ds(r, CHUNK)], z)
    pltpu.sync_copy(z, acc_out.at[cid, pl.ds(r, CHUNK)])
  r = row0 + n_full * CHUNK
  pltpu.sync_copy(acc_s.at[pl.ds(r, rem)], z.at[pl.ds(0, rem)])
  pltpu.sync_copy(z.at[pl.ds(0, rem)], acc_out.at[cid, pl.ds(r, rem)])

  @pl.when(sid == NS - 1)
  def _():
    pltpu.sync_copy(acc_s.at[pl.ds(NS * RPS, TAIL)], z.at[pl.ds(0, TAIL)])
    pltpu.sync_copy(z.at[pl.ds(0, TAIL)],
                    acc_out.at[cid, pl.ds(NS * RPS, TAIL)])


_agg = pl.kernel(
    _agg_body,
    out_type=jax.ShapeDtypeStruct((NC, N, D), _F32),
    mesh=_MESH,
    scratch_types=[
        pltpu.VMEM((4 * NBUF, CHUNK), jnp.int32),
        pltpu.VMEM((4 * NBUF, CHUNK), jnp.int32),
        pltpu.VMEM((NBUF, CHUNK, D), _F32),
        pltpu.VMEM_SHARED((N, D), _F32),
    ] + [pltpu.SemaphoreType.DMA] * (2 * NBUF + 8),
)


CNBUF = 8
C2 = NCHUNK // CNBUF


def _cnt_body(dst_hbm, cnt_out, dst_v, ones_v, cnt_s, *sems):
  cid = lax.axis_index("c")
  sid = lax.axis_index("s")
  wid = sid * NC + cid

  @pl.loop(0, CHUNK)
  def _(i):
    @pl.loop(0, D // 16)
    def _(j):
      ones_v[i, pl.ds(j * 16, 16)] = jnp.zeros((16,), _F32)

  n_full = RPS // CHUNK
  rem = RPS - n_full * CHUNK
  row0 = sid * RPS

  @pl.loop(0, n_full)
  def _(k):
    pltpu.sync_copy(ones_v, cnt_s.at[pl.ds(row0 + k * CHUNK, CHUNK)])
  pltpu.sync_copy(ones_v.at[pl.ds(0, rem)],
                  cnt_s.at[pl.ds(row0 + n_full * CHUNK, rem)])

  @pl.when(sid == NS - 1)
  def _():
    pltpu.sync_copy(ones_v.at[pl.ds(0, TAIL)], cnt_s.at[pl.ds(NS * RPS, TAIL)])

  @pl.loop(0, CHUNK)
  def _(i):
    @pl.loop(0, D // 16)
    def _(j):
      ones_v[i, pl.ds(j * 16, 16)] = jnp.full((16,), 1.0, _F32)

  plsc.subcore_barrier()

  base = wid * EPW

  @pl.loop(0, C2)
  def _(k2):
    for b in range(CNBUF):
      @pl.when(k2 > 0)
      def _():
        pltpu.make_async_copy(ones_v, cnt_s.at[dst_v.at[b]], sems[b]).wait()
      off = base + (k2 * CNBUF + b) * CHUNK
      pltpu.sync_copy(dst_hbm.at[pl.ds(off, CHUNK)], dst_v.at[b])
      pltpu.async_copy(ones_v, cnt_s.at[dst_v.at[b]], sems[b], add=True)

  for b in range(CNBUF):
    pltpu.make_async_copy(ones_v, cnt_s.at[dst_v.at[b]], sems[b]).wait()

  for t in range(C2 * CNBUF, NCHUNK):
    off = base + t * CHUNK
    pltpu.sync_copy(dst_hbm.at[pl.ds(off, CHUNK)], dst_v.at[0])
    pltpu.sync_copy(ones_v, cnt_s.at[dst_v.at[0]], add=True)

  plsc.subcore_barrier()

  @pl.loop(0, n_full)
  def _(k):
    r = row0 + k * CHUNK
    pltpu.sync_copy(cnt_s.at[pl.ds(r, CHUNK)], ones_v)
    pltpu.sync_copy(ones_v, cnt_out.at[cid, pl.ds(r, CHUNK)])
  r = row0 + n_full * CHUNK
  pltpu.sync_copy(cnt_s.at[pl.ds(r, rem)], ones_v.at[pl.ds(0, rem)])
  pltpu.sync_copy(ones_v.at[pl.ds(0, rem)], cnt_out.at[cid, pl.ds(r, rem)])

  @pl.when(sid == NS - 1)
  def _():
    pltpu.sync_copy(cnt_s.at[pl.ds(NS * RPS, TAIL)], ones_v.at[pl.ds(0, TAIL)])
    pltpu.sync_copy(ones_v.at[pl.ds(0, TAIL)],
                    cnt_out.at[cid, pl.ds(NS * RPS, TAIL)])


_counts = pl.kernel(
    _cnt_body,
    out_type=jax.ShapeDtypeStruct((NC, N, D), _F32),
    mesh=_MESH,
    scratch_types=[
        pltpu.VMEM((CNBUF, CHUNK), jnp.int32),
        pltpu.VMEM((CHUNK, D), _F32),
        pltpu.VMEM_SHARED((N, D), _F32),
    ] + [pltpu.SemaphoreType.DMA] * CNBUF,
)


def _dot_t(a, w):
  return lax.dot_general(a, w, (((1,), (1,)), ((), ())),
                         precision=lax.Precision.HIGHEST,
                         preferred_element_type=_F32)


_BR = 1000


def _layer_tc(acc, rec, h, Wl, bl, Wr, relu):
  def body(acc_ref, rec_ref, h_ref, wl_ref, bl_ref, wr_ref, out_ref):
    mean = (acc_ref[0] + acc_ref[1]) * rec_ref[...]
    y = _dot_t(mean, wl_ref[...]) + _dot_t(h_ref[...], wr_ref[...]) + bl_ref[...]
    out_ref[...] = jnp.maximum(y, 0.0) if relu else y

  return pl.pallas_call(
      body,
      grid=(N // _BR,),
      in_specs=[
          pl.BlockSpec((NC, _BR, D), lambda i: (0, i, 0)),
          pl.BlockSpec((_BR, 1), lambda i: (i, 0)),
          pl.BlockSpec((_BR, D), lambda i: (i, 0)),
          pl.BlockSpec((D, D), lambda i: (0, 0)),
          pl.BlockSpec((1, D), lambda i: (0, 0)),
          pl.BlockSpec((D, D), lambda i: (0, 0)),
      ],
      out_specs=pl.BlockSpec((_BR, D), lambda i: (i, 0)),
      out_shape=jax.ShapeDtypeStruct((N, D), _F32),
  )(acc, rec, h, Wl, bl, Wr)


def _recip_tc(cnt):
  def body(cnt_ref, out_ref):
    c = cnt_ref[0, :, 0:1] + cnt_ref[1, :, 0:1]
    out_ref[...] = 1.0 / jnp.maximum(c, 1.0)

  return pl.pallas_call(
      body,
      grid=(N // _BR,),
      in_specs=[pl.BlockSpec((NC, _BR, D), lambda i: (0, i, 0))],
      out_specs=pl.BlockSpec((_BR, 1), lambda i: (i, 0)),
      out_shape=jax.ShapeDtypeStruct((N, 1), _F32),
  )(cnt)


def _last_layer_tc(acc, rec, h, Wl, bl, Wr, Wp, bp):
  g = N // _BR

  def body(acc_ref, rec_ref, h_ref, wl_ref, bl_ref, wr_ref, wp_ref, bp_ref,
           h3_ref, proj_ref, stats_ref, ssum, ssq):
    i = pl.program_id(0)
    mean = (acc_ref[0] + acc_ref[1]) * rec_ref[...]
    y = _dot_t(mean, wl_ref[...]) + _dot_t(h_ref[...], wr_ref[...]) + bl_ref[...]
    h3_ref[...] = y
    p = _dot_t(y, wp_ref[...]) + bp_ref[...]
    proj_ref[...] = p

    @pl.when(i == 0)
    def _():
      ssum[...] = jnp.zeros_like(ssum)
      ssq[...] = jnp.zeros_like(ssq)

    ssum[...] += jnp.sum(p, axis=0, keepdims=True)
    ssq[...] += jnp.sum(p * p, axis=0, keepdims=True)

    @pl.when(i == g - 1)
    def _():
      stats_ref[0:1, :] = ssum[...]
      stats_ref[1:2, :] = ssq[...]

  return pl.pallas_call(
      body,
      grid=(g,),
      in_specs=[
          pl.BlockSpec((NC, _BR, D), lambda i: (0, i, 0)),
          pl.BlockSpec((_BR, 1), lambda i: (i, 0)),
          pl.BlockSpec((_BR, D), lambda i: (i, 0)),
          pl.BlockSpec((D, D), lambda i: (0, 0)),
          pl.BlockSpec((1, D), lambda i: (0, 0)),
          pl.BlockSpec((D, D), lambda i: (0, 0)),
          pl.BlockSpec((D, D), lambda i: (0, 0)),
          pl.BlockSpec((1, D), lambda i: (0, 0)),
      ],
      out_specs=[
          pl.BlockSpec((_BR, D), lambda i: (i, 0)),
          pl.BlockSpec((_BR, D), lambda i: (i, 0)),
          pl.BlockSpec((2, D), lambda i: (0, 0)),
      ],
      out_shape=[
          jax.ShapeDtypeStruct((N, D), _F32),
          jax.ShapeDtypeStruct((N, D), _F32),
          jax.ShapeDtypeStruct((2, D), _F32),
      ],
      scratch_shapes=[pltpu.VMEM((1, D), _F32), pltpu.VMEM((1, D), _F32)],
  )(acc, rec, h, Wl, bl, Wr, Wp, bp)


def _bnorm_tc(proj, stats, gamma, beta):
  def body(proj_ref, stats_ref, g_ref, b_ref, out_ref):
    inv_n = 1.0 / N
    mu = stats_ref[0:1, :] * inv_n
    var = stats_ref[1:2, :] * inv_n - mu * mu
    scale = g_ref[...] * lax.rsqrt(var + 1e-5)
    out_ref[...] = proj_ref[...] * scale + (b_ref[...] - mu * scale)

  return pl.pallas_call(
      body,
      grid=(N // _BR,),
      in_specs=[
          pl.BlockSpec((_BR, D), lambda i: (i, 0)),
          pl.BlockSpec((2, D), lambda i: (0, 0)),
          pl.BlockSpec((1, D), lambda i: (0, 0)),
          pl.BlockSpec((1, D), lambda i: (0, 0)),
      ],
      out_specs=pl.BlockSpec((_BR, D), lambda i: (i, 0)),
      out_shape=jax.ShapeDtypeStruct((N, D), _F32),
  )(proj, stats, gamma, beta)


def kernel(x, edge_index, Wl0, bl0, Wr0, Wl1, bl1, Wr1, Wl2, bl2, Wr2,
           Wp, bp, gamma, beta):
  src = edge_index[0]
  dst = edge_index[1]
  bl0r = bl0.reshape(1, D)
  bl1r = bl1.reshape(1, D)
  bl2r = bl2.reshape(1, D)
  bpr = bp.reshape(1, D)
  gr = gamma.reshape(1, D)
  br = beta.reshape(1, D)

  cnt = _counts(dst)
  rec = _recip_tc(cnt)
  acc0 = _agg(x, src, dst)
  h1 = _layer_tc(acc0, rec, x, Wl0, bl0r, Wr0, True)
  acc1 = _agg(h1, src, dst)
  h2 = _layer_tc(acc1, rec, h1, Wl1, bl1r, Wr1, True)
  acc2 = _agg(h2, src, dst)
  h3, proj, stats = _last_layer_tc(acc2, rec, h2, Wl2, bl2r, Wr2, Wp, bpr)
  h_out = _bnorm_tc(proj, stats, gr, br)
  return (h3, h_out)
